# baseline (speedup 1.0000x reference)
import jax
import jax.numpy as jnp
from jax.experimental import pallas as pl
from jax.experimental.pallas import tpu as pltpu

_EPS = 1e-5
_SLOPE = 0.1
_MM = jnp.bfloat16


def _conv_stats_from_patches(acc, y_ref, s_ref, q_ref):
    y_ref[0] = acc.reshape(y_ref.shape[1:]).astype(y_ref.dtype)
    s_ref[0] = jnp.sum(acc, axis=0, keepdims=True)
    q_ref[0] = jnp.sum(acc * acc, axis=0, keepdims=True)


def _conv1_kernel(xc_ref, w_ref, y_ref, s_ref, q_ref):
    H = xc_ref.shape[1] - 2
    W = xc_ref.shape[2]
    K = xc_ref.shape[3]
    acc = None
    for dy in range(3):
        slab = xc_ref[0, dy:dy + H].reshape(H * W, K)
        d = jnp.dot(slab, w_ref[dy], preferred_element_type=jnp.float32)
        acc = d if acc is None else acc + d
    _conv_stats_from_patches(acc, y_ref, s_ref, q_ref)


def _mid_kernel(y1_ref, s_ref, q_ref, g_ref, b_ref, w_ref,
                y2_ref, s2_ref, q2_ref, hc_ref):
    N = s_ref.shape[0]
    H, W, C1 = y1_ref.shape[1], y1_ref.shape[2], y1_ref.shape[3]
    R = float(N * H * W)

    mean = jnp.sum(s_ref[...], axis=0) / R
    var = jnp.sum(q_ref[...], axis=0) / R - mean * mean
    scale = g_ref[...] * jax.lax.rsqrt(var + _EPS)
    shift = b_ref[...] - mean * scale

    h = (y1_ref[0].astype(jnp.float32) * scale.reshape(1, 1, C1)
         + shift.reshape(1, 1, C1))
    h = jnp.where(h >= 0.0, h, _SLOPE * h)
    hp = jnp.pad(h, ((0, 0), (1, 1), (0, 0))).astype(_MM)

    hc_ref[0] = jnp.zeros((W, 3 * C1), _MM)
    hc_ref[H + 1] = jnp.zeros((W, 3 * C1), _MM)
    for dx in range(3):
        hc_ref[1:H + 1, :, dx * C1:(dx + 1) * C1] = hp[:, dx:dx + W, :]

    acc = None
    for dy in range(3):
        slab = hc_ref[dy:dy + H].reshape(H * W, 3 * C1)
        d = jnp.dot(slab, w_ref[dy], preferred_element_type=jnp.float32)
        acc = d if acc is None else acc + d
    _conv_stats_from_patches(acc, y2_ref, s2_ref, q2_ref)


def _out_kernel(y2_ref, s_ref, q_ref, g_ref, b_ref, ident_ref, o_ref):
    N = s_ref.shape[0]
    M, C = y2_ref.shape[1], y2_ref.shape[2]
    R = float(N * M)
    mean = jnp.sum(s_ref[...], axis=0) / R
    var = jnp.sum(q_ref[...], axis=0) / R - mean * mean
    scale = g_ref[...] * jax.lax.rsqrt(var + _EPS)
    shift = b_ref[...] - mean * scale
    h = y2_ref[0].astype(jnp.float32) * scale + shift
    h = jnp.where(h >= 0.0, h, _SLOPE * h)
    o_ref[0] = jax.lax.dot_general(
        ident_ref[...], h, (((1,), (1,)), ((), ())),
        preferred_element_type=jnp.float32)


@jax.jit
def _double_conv(x_nchw, w1, g1, b1, w2, g2, b2):
    N, C1, H, W = x_nchw.shape
    Cm = w1.shape[-1]
    C2 = w2.shape[-1]

    x = jnp.transpose(x_nchw, (0, 2, 3, 1)).astype(_MM)
    xp = jnp.pad(x, ((0, 0), (1, 1), (1, 1), (0, 0)))
    xc = jnp.concatenate(
        [xp[:, :, 0:W], xp[:, :, 1:W + 1], xp[:, :, 2:W + 2]], axis=-1)
    w1c = w1.reshape(3, 3 * C1, Cm).astype(_MM)
    w2c = w2.reshape(3, 3 * Cm, C2).astype(_MM)
    g1r = g1.astype(jnp.float32).reshape(1, Cm)
    b1r = b1.astype(jnp.float32).reshape(1, Cm)
    g2r = g2.astype(jnp.float32).reshape(1, C2)
    b2r = b2.astype(jnp.float32).reshape(1, C2)

    vec = lambda c: pl.BlockSpec((1, c), lambda i: (0, 0))
    stat_in = lambda c: pl.BlockSpec((N, 1, c), lambda i: (0, 0, 0))
    stat_out = lambda c: pl.BlockSpec((1, 1, c), lambda i: (i, 0, 0))
    img = lambda c: pl.BlockSpec((1, H, W, c), lambda i: (i, 0, 0, 0))

    y1, s1, q1 = pl.pallas_call(
        _conv1_kernel,
        grid=(N,),
        in_specs=[pl.BlockSpec((1, H + 2, W, 3 * C1), lambda i: (i, 0, 0, 0)),
                  pl.BlockSpec((3, 3 * C1, Cm), lambda i: (0, 0, 0))],
        out_specs=[img(Cm), stat_out(Cm), stat_out(Cm)],
        out_shape=[jax.ShapeDtypeStruct((N, H, W, Cm), _MM),
                   jax.ShapeDtypeStruct((N, 1, Cm), jnp.float32),
                   jax.ShapeDtypeStruct((N, 1, Cm), jnp.float32)],
        compiler_params=pltpu.CompilerParams(
            dimension_semantics=("parallel",)),
    )(xc, w1c)

    y2, s2, q2 = pl.pallas_call(
        _mid_kernel,
        grid=(N,),
        in_specs=[img(Cm), stat_in(Cm), stat_in(Cm), vec(Cm), vec(Cm),
                  pl.BlockSpec((3, 3 * Cm, C2), lambda i: (0, 0, 0))],
        out_specs=[pl.BlockSpec((1, H * W, C2), lambda i: (i, 0, 0)),
                   stat_out(C2), stat_out(C2)],
        out_shape=[jax.ShapeDtypeStruct((N, H * W, C2), _MM),
                   jax.ShapeDtypeStruct((N, 1, C2), jnp.float32),
                   jax.ShapeDtypeStruct((N, 1, C2), jnp.float32)],
        scratch_shapes=[pltpu.VMEM((H + 2, W, 3 * Cm), _MM)],
        compiler_params=pltpu.CompilerParams(
            dimension_semantics=("parallel",)),
    )(y1, s1, q1, g1r, b1r, w2c)

    ident = jnp.eye(C2, dtype=jnp.float32)
    out = pl.pallas_call(
        _out_kernel,
        grid=(N,),
        in_specs=[pl.BlockSpec((1, H * W, C2), lambda i: (i, 0, 0)),
                  stat_in(C2), stat_in(C2), vec(C2), vec(C2),
                  pl.BlockSpec((C2, C2), lambda i: (0, 0))],
        out_specs=pl.BlockSpec((1, C2, H * W), lambda i: (i, 0, 0)),
        out_shape=jax.ShapeDtypeStruct((N, C2, H * W), jnp.float32),
        compiler_params=pltpu.CompilerParams(
            dimension_semantics=("parallel",)),
    )(y2, s2, q2, g2r, b2r, ident)

    return out.reshape(N, C2, H, W)


def kernel(x_nchw, w1, g1, b1, w2, g2, b2):
    return _double_conv(x_nchw, w1, g1, b1, w2, g2, b2)


# speedup vs baseline: 1.1571x; 1.1571x over previous
import jax
import jax.numpy as jnp
from jax.experimental import pallas as pl
from jax.experimental.pallas import tpu as pltpu

_EPS = 1e-5
_SLOPE = 0.1
_MM = jnp.bfloat16


def _conv_stats_from_patches(acc, y_ref, s_ref, q_ref):
    y_ref[0] = acc.reshape(y_ref.shape[1:]).astype(y_ref.dtype)
    s_ref[0] = jnp.sum(acc, axis=0, keepdims=True)
    q_ref[0] = jnp.sum(acc * acc, axis=0, keepdims=True)


def _conv1_kernel(xc_ref, w_ref, y_ref, s_ref, q_ref):
    H = xc_ref.shape[1] - 2
    W = xc_ref.shape[2]
    K = xc_ref.shape[3]
    acc = None
    for dy in range(3):
        slab = xc_ref[0, dy:dy + H].reshape(H * W, K)
        d = jnp.dot(slab, w_ref[dy], preferred_element_type=jnp.float32)
        acc = d if acc is None else acc + d
    _conv_stats_from_patches(acc, y_ref, s_ref, q_ref)


def _mid_kernel(y1_ref, s_ref, q_ref, g_ref, b_ref, w_ref,
                y2_ref, s2_ref, q2_ref, hc_ref):
    N = s_ref.shape[0]
    H, W, C1 = y1_ref.shape[1], y1_ref.shape[2], y1_ref.shape[3]
    R = float(N * H * W)

    mean = jnp.sum(s_ref[...], axis=0) / R
    var = jnp.sum(q_ref[...], axis=0) / R - mean * mean
    scale = g_ref[...] * jax.lax.rsqrt(var + _EPS)
    shift = b_ref[...] - mean * scale

    h = (y1_ref[0].astype(jnp.float32) * scale.reshape(1, 1, C1)
         + shift.reshape(1, 1, C1))
    h = jnp.where(h >= 0.0, h, _SLOPE * h)
    hp = jnp.pad(h, ((0, 0), (1, 1), (0, 0))).astype(_MM)

    hc_ref[0] = jnp.zeros((W, 3 * C1), _MM)
    hc_ref[H + 1] = jnp.zeros((W, 3 * C1), _MM)
    for dx in range(3):
        hc_ref[1:H + 1, :, dx * C1:(dx + 1) * C1] = hp[:, dx:dx + W, :]

    acc = None
    for dy in range(3):
        slab = hc_ref[dy:dy + H].reshape(H * W, 3 * C1)
        d = jnp.dot(slab, w_ref[dy], preferred_element_type=jnp.float32)
        acc = d if acc is None else acc + d
    _conv_stats_from_patches(acc, y2_ref, s2_ref, q2_ref)


def _out_kernel(y2_ref, s_ref, q_ref, g_ref, b_ref, ident_ref, o_ref):
    N = s_ref.shape[0]
    M, C = y2_ref.shape[1], y2_ref.shape[2]
    R = float(N * M)
    mean = jnp.sum(s_ref[...], axis=0) / R
    var = jnp.sum(q_ref[...], axis=0) / R - mean * mean
    scale = g_ref[...] * jax.lax.rsqrt(var + _EPS)
    shift = b_ref[...] - mean * scale
    h = y2_ref[0].astype(jnp.float32) * scale + shift
    h = jnp.where(h >= 0.0, h, _SLOPE * h)
    del ident_ref
    o_ref[0] = h


@jax.jit
def _double_conv(x_nchw, w1, g1, b1, w2, g2, b2):
    N, C1, H, W = x_nchw.shape
    Cm = w1.shape[-1]
    C2 = w2.shape[-1]

    x = jnp.transpose(x_nchw, (0, 2, 3, 1)).astype(_MM)
    xp = jnp.pad(x, ((0, 0), (1, 1), (1, 1), (0, 0)))
    xc = jnp.concatenate(
        [xp[:, :, 0:W], xp[:, :, 1:W + 1], xp[:, :, 2:W + 2]], axis=-1)
    w1c = w1.reshape(3, 3 * C1, Cm).astype(_MM)
    w2c = w2.reshape(3, 3 * Cm, C2).astype(_MM)
    g1r = g1.astype(jnp.float32).reshape(1, Cm)
    b1r = b1.astype(jnp.float32).reshape(1, Cm)
    g2r = g2.astype(jnp.float32).reshape(1, C2)
    b2r = b2.astype(jnp.float32).reshape(1, C2)

    vec = lambda c: pl.BlockSpec((1, c), lambda i: (0, 0))
    stat_in = lambda c: pl.BlockSpec((N, 1, c), lambda i: (0, 0, 0))
    stat_out = lambda c: pl.BlockSpec((1, 1, c), lambda i: (i, 0, 0))
    img = lambda c: pl.BlockSpec((1, H, W, c), lambda i: (i, 0, 0, 0))

    y1, s1, q1 = pl.pallas_call(
        _conv1_kernel,
        grid=(N,),
        in_specs=[pl.BlockSpec((1, H + 2, W, 3 * C1), lambda i: (i, 0, 0, 0)),
                  pl.BlockSpec((3, 3 * C1, Cm), lambda i: (0, 0, 0))],
        out_specs=[img(Cm), stat_out(Cm), stat_out(Cm)],
        out_shape=[jax.ShapeDtypeStruct((N, H, W, Cm), _MM),
                   jax.ShapeDtypeStruct((N, 1, Cm), jnp.float32),
                   jax.ShapeDtypeStruct((N, 1, Cm), jnp.float32)],
        compiler_params=pltpu.CompilerParams(
            dimension_semantics=("parallel",)),
    )(xc, w1c)

    y2, s2, q2 = pl.pallas_call(
        _mid_kernel,
        grid=(N,),
        in_specs=[img(Cm), stat_in(Cm), stat_in(Cm), vec(Cm), vec(Cm),
                  pl.BlockSpec((3, 3 * Cm, C2), lambda i: (0, 0, 0))],
        out_specs=[pl.BlockSpec((1, H * W, C2), lambda i: (i, 0, 0)),
                   stat_out(C2), stat_out(C2)],
        out_shape=[jax.ShapeDtypeStruct((N, H * W, C2), _MM),
                   jax.ShapeDtypeStruct((N, 1, C2), jnp.float32),
                   jax.ShapeDtypeStruct((N, 1, C2), jnp.float32)],
        scratch_shapes=[pltpu.VMEM((H + 2, W, 3 * Cm), _MM)],
        compiler_params=pltpu.CompilerParams(
            dimension_semantics=("parallel",)),
    )(y1, s1, q1, g1r, b1r, w2c)

    ident = jnp.eye(C2, dtype=jnp.float32)
    out = pl.pallas_call(
        _out_kernel,
        grid=(N,),
        in_specs=[pl.BlockSpec((1, H * W, C2), lambda i: (i, 0, 0)),
                  stat_in(C2), stat_in(C2), vec(C2), vec(C2),
                  pl.BlockSpec((C2, C2), lambda i: (0, 0))],
        out_specs=pl.BlockSpec((1, H * W, C2), lambda i: (i, 0, 0)),
        out_shape=jax.ShapeDtypeStruct((N, H * W, C2), jnp.float32),
        compiler_params=pltpu.CompilerParams(
            dimension_semantics=("parallel",)),
    )(y2, s2, q2, g2r, b2r, ident)

    return jnp.transpose(out.reshape(N, H, W, C2), (0, 3, 1, 2))


def kernel(x_nchw, w1, g1, b1, w2, g2, b2):
    return _double_conv(x_nchw, w1, g1, b1, w2, g2, b2)


# speedup vs baseline: 1.4637x; 1.2650x over previous
import jax
import jax.numpy as jnp
from jax.experimental import pallas as pl
from jax.experimental.pallas import tpu as pltpu

_EPS = 1e-5
_SLOPE = 0.1
_MM = jnp.bfloat16


def _conv_stats_from_patches(acc, y_ref, s_ref, q_ref):
    y_ref[0] = acc.reshape(y_ref.shape[1:]).astype(y_ref.dtype)
    s_ref[0] = jnp.sum(acc, axis=0, keepdims=True)
    q_ref[0] = jnp.sum(acc * acc, axis=0, keepdims=True)


def _conv1_kernel(xp_ref, w_ref, y_ref, s_ref, q_ref, xc_ref):
    H = xp_ref.shape[1] - 2
    W = xp_ref.shape[2] - 2
    C1 = xp_ref.shape[3]
    for dx in range(3):
        xc_ref[:, :, dx * C1:(dx + 1) * C1] = xp_ref[0, :, dx:dx + W, :]
    acc = None
    for dy in range(3):
        slab = xc_ref[dy:dy + H].reshape(H * W, 3 * C1)
        d = jnp.dot(slab, w_ref[dy], preferred_element_type=jnp.float32)
        acc = d if acc is None else acc + d
    _conv_stats_from_patches(acc, y_ref, s_ref, q_ref)


def _mid_kernel(y1_ref, s_ref, q_ref, g_ref, b_ref, w_ref,
                y2_ref, s2_ref, q2_ref, hc_ref):
    N = s_ref.shape[0]
    H, W, C1 = y1_ref.shape[1], y1_ref.shape[2], y1_ref.shape[3]
    R = float(N * H * W)

    mean = jnp.sum(s_ref[...], axis=0) / R
    var = jnp.sum(q_ref[...], axis=0) / R - mean * mean
    scale = g_ref[...] * jax.lax.rsqrt(var + _EPS)
    shift = b_ref[...] - mean * scale

    h = (y1_ref[0].astype(jnp.float32) * scale.reshape(1, 1, C1)
         + shift.reshape(1, 1, C1))
    h = jnp.where(h >= 0.0, h, _SLOPE * h)
    hp = jnp.pad(h, ((0, 0), (1, 1), (0, 0))).astype(_MM)

    hc_ref[0] = jnp.zeros((W, 3 * C1), _MM)
    hc_ref[H + 1] = jnp.zeros((W, 3 * C1), _MM)
    for dx in range(3):
        hc_ref[1:H + 1, :, dx * C1:(dx + 1) * C1] = hp[:, dx:dx + W, :]

    acc = None
    for dy in range(3):
        slab = hc_ref[dy:dy + H].reshape(H * W, 3 * C1)
        d = jnp.dot(slab, w_ref[dy], preferred_element_type=jnp.float32)
        acc = d if acc is None else acc + d
    _conv_stats_from_patches(acc, y2_ref, s2_ref, q2_ref)


def _out_kernel(y2_ref, s_ref, q_ref, g_ref, b_ref, ident_ref, o_ref):
    N = s_ref.shape[0]
    M, C = y2_ref.shape[1], y2_ref.shape[2]
    R = float(N * M)
    mean = jnp.sum(s_ref[...], axis=0) / R
    var = jnp.sum(q_ref[...], axis=0) / R - mean * mean
    scale = g_ref[...] * jax.lax.rsqrt(var + _EPS)
    shift = b_ref[...] - mean * scale
    h = y2_ref[0].astype(jnp.float32) * scale + shift
    h = jnp.where(h >= 0.0, h, _SLOPE * h)
    del ident_ref
    o_ref[0] = h


@jax.jit
def _double_conv(x_nchw, w1, g1, b1, w2, g2, b2):
    N, C1, H, W = x_nchw.shape
    Cm = w1.shape[-1]
    C2 = w2.shape[-1]

    x = jnp.transpose(x_nchw, (0, 2, 3, 1)).astype(_MM)
    xp = jnp.pad(x, ((0, 0), (1, 1), (1, 1), (0, 0)))
    w1c = w1.reshape(3, 3 * C1, Cm).astype(_MM)
    w2c = w2.reshape(3, 3 * Cm, C2).astype(_MM)
    g1r = g1.astype(jnp.float32).reshape(1, Cm)
    b1r = b1.astype(jnp.float32).reshape(1, Cm)
    g2r = g2.astype(jnp.float32).reshape(1, C2)
    b2r = b2.astype(jnp.float32).reshape(1, C2)

    vec = lambda c: pl.BlockSpec((1, c), lambda i: (0, 0))
    stat_in = lambda c: pl.BlockSpec((N, 1, c), lambda i: (0, 0, 0))
    stat_out = lambda c: pl.BlockSpec((1, 1, c), lambda i: (i, 0, 0))
    img = lambda c: pl.BlockSpec((1, H, W, c), lambda i: (i, 0, 0, 0))

    y1, s1, q1 = pl.pallas_call(
        _conv1_kernel,
        grid=(N,),
        in_specs=[pl.BlockSpec((1, H + 2, W + 2, C1), lambda i: (i, 0, 0, 0)),
                  pl.BlockSpec((3, 3 * C1, Cm), lambda i: (0, 0, 0))],
        out_specs=[img(Cm), stat_out(Cm), stat_out(Cm)],
        out_shape=[jax.ShapeDtypeStruct((N, H, W, Cm), _MM),
                   jax.ShapeDtypeStruct((N, 1, Cm), jnp.float32),
                   jax.ShapeDtypeStruct((N, 1, Cm), jnp.float32)],
        scratch_shapes=[pltpu.VMEM((H + 2, W, 3 * C1), _MM)],
        compiler_params=pltpu.CompilerParams(
            dimension_semantics=("parallel",)),
    )(xp, w1c)

    y2, s2, q2 = pl.pallas_call(
        _mid_kernel,
        grid=(N,),
        in_specs=[img(Cm), stat_in(Cm), stat_in(Cm), vec(Cm), vec(Cm),
                  pl.BlockSpec((3, 3 * Cm, C2), lambda i: (0, 0, 0))],
        out_specs=[pl.BlockSpec((1, H * W, C2), lambda i: (i, 0, 0)),
                   stat_out(C2), stat_out(C2)],
        out_shape=[jax.ShapeDtypeStruct((N, H * W, C2), _MM),
                   jax.ShapeDtypeStruct((N, 1, C2), jnp.float32),
                   jax.ShapeDtypeStruct((N, 1, C2), jnp.float32)],
        scratch_shapes=[pltpu.VMEM((H + 2, W, 3 * Cm), _MM)],
        compiler_params=pltpu.CompilerParams(
            dimension_semantics=("parallel",)),
    )(y1, s1, q1, g1r, b1r, w2c)

    ident = jnp.eye(C2, dtype=jnp.float32)
    out = pl.pallas_call(
        _out_kernel,
        grid=(N,),
        in_specs=[pl.BlockSpec((1, H * W, C2), lambda i: (i, 0, 0)),
                  stat_in(C2), stat_in(C2), vec(C2), vec(C2),
                  pl.BlockSpec((C2, C2), lambda i: (0, 0))],
        out_specs=pl.BlockSpec((1, H * W, C2), lambda i: (i, 0, 0)),
        out_shape=jax.ShapeDtypeStruct((N, H * W, C2), jnp.float32),
        compiler_params=pltpu.CompilerParams(
            dimension_semantics=("parallel",)),
    )(y2, s2, q2, g2r, b2r, ident)

    return jnp.transpose(out.reshape(N, H, W, C2), (0, 3, 1, 2))


def kernel(x_nchw, w1, g1, b1, w2, g2, b2):
    return _double_conv(x_nchw, w1, g1, b1, w2, g2, b2)


# speedup vs baseline: 1.5171x; 1.0365x over previous
import jax
import jax.numpy as jnp
from jax.experimental import pallas as pl
from jax.experimental.pallas import tpu as pltpu

_EPS = 1e-5
_SLOPE = 0.1
_MM = jnp.bfloat16


def _tap_matmuls(cat_ref, w_ref, H, W, C):
    acc = None
    for dy in range(3):
        slab = cat_ref[dy:dy + H].reshape(H * W, 3 * C)
        d = jnp.dot(slab, w_ref[dy], preferred_element_type=jnp.float32)
        acc = d if acc is None else acc + d
    return acc


def _scale_shift(st_ref, g_ref, b_ref, r_total):
    mean = st_ref[0:1] / r_total
    var = st_ref[1:2] / r_total - mean * mean
    scale = g_ref[...] * jax.lax.rsqrt(var + _EPS)
    shift = b_ref[...] - mean * scale
    return scale, shift


def _fused_kernel(xp_ref, w1_ref, w2_ref, g1_ref, b1_ref, g2_ref, b2_ref,
                  o_ref,
                  y1_ref, y2_ref, st1_ref, st2_ref, xc_ref, hc_ref, hp_ref):
    p = pl.program_id(0)
    i = pl.program_id(1)
    N = pl.num_programs(1)
    H = y1_ref.shape[1]
    W = y1_ref.shape[2]
    C1 = xp_ref.shape[3]
    Cm = y1_ref.shape[3]
    C2 = y2_ref.shape[2]
    R = float(N * H * W)

    @pl.when(p == 0)
    def _phase0():
        @pl.when(i == 0)
        def _():
            st1_ref[...] = jnp.zeros_like(st1_ref)
            st2_ref[...] = jnp.zeros_like(st2_ref)
            hc_ref[0] = jnp.zeros((W, 3 * Cm), _MM)
            hc_ref[H + 1] = jnp.zeros((W, 3 * Cm), _MM)
        for dx in range(3):
            xc_ref[:, :, dx * C1:(dx + 1) * C1] = xp_ref[0, :, dx:dx + W, :]
        acc = _tap_matmuls(xc_ref, w1_ref, H, W, C1)
        y1_ref[pl.ds(i, 1)] = acc.reshape(1, H, W, Cm).astype(_MM)
        st1_ref[0:1] = st1_ref[0:1] + jnp.sum(acc, axis=0, keepdims=True)
        st1_ref[1:2] = st1_ref[1:2] + jnp.sum(acc * acc, axis=0,
                                              keepdims=True)

    @pl.when(p == 1)
    def _phase1():
        scale, shift = _scale_shift(st1_ref, g1_ref, b1_ref, R)
        h = (y1_ref[pl.ds(i, 1)][0].astype(jnp.float32)
             * scale.reshape(1, 1, Cm) + shift.reshape(1, 1, Cm))
        h = jnp.where(h >= 0.0, h, _SLOPE * h)
        hp_ref[:, 1:W + 1, :] = h.astype(_MM)
        @pl.when(i == 0)
        def _():
            hp_ref[:, 0:1, :] = jnp.zeros((H, 1, Cm), _MM)
            hp_ref[:, W + 1:W + 2, :] = jnp.zeros((H, 1, Cm), _MM)
        for dx in range(3):
            hc_ref[1:H + 1, :, dx * Cm:(dx + 1) * Cm] = hp_ref[:, dx:dx + W, :]
        acc = _tap_matmuls(hc_ref, w2_ref, H, W, Cm)
        y2_ref[pl.ds(i, 1)] = acc.reshape(1, H * W, C2).astype(_MM)
        st2_ref[0:1] = st2_ref[0:1] + jnp.sum(acc, axis=0, keepdims=True)
        st2_ref[1:2] = st2_ref[1:2] + jnp.sum(acc * acc, axis=0,
                                              keepdims=True)

    @pl.when(p == 2)
    def _phase2():
        scale, shift = _scale_shift(st2_ref, g2_ref, b2_ref, R)
        h = y2_ref[pl.ds(i, 1)][0].astype(jnp.float32) * scale + shift
        o_ref[0] = jnp.where(h >= 0.0, h, _SLOPE * h)


@jax.jit
def _double_conv(x_nchw, w1, g1, b1, w2, g2, b2):
    N, C1, H, W = x_nchw.shape
    Cm = w1.shape[-1]
    C2 = w2.shape[-1]

    x = jnp.transpose(x_nchw, (0, 2, 3, 1)).astype(_MM)
    xp = jnp.pad(x, ((0, 0), (1, 1), (1, 1), (0, 0)))
    w1c = w1.reshape(3, 3 * C1, Cm).astype(_MM)
    w2c = w2.reshape(3, 3 * Cm, C2).astype(_MM)
    g1r = g1.astype(jnp.float32).reshape(1, Cm)
    b1r = b1.astype(jnp.float32).reshape(1, Cm)
    g2r = g2.astype(jnp.float32).reshape(1, C2)
    b2r = b2.astype(jnp.float32).reshape(1, C2)

    vec = lambda c: pl.BlockSpec((1, c), lambda p, i: (0, 0))

    out = pl.pallas_call(
        _fused_kernel,
        grid=(3, N),
        in_specs=[
            pl.BlockSpec((1, H + 2, W + 2, C1),
                         lambda p, i: (jnp.where(p == 0, i, N - 1), 0, 0, 0)),
            pl.BlockSpec((3, 3 * C1, Cm), lambda p, i: (0, 0, 0)),
            pl.BlockSpec((3, 3 * Cm, C2), lambda p, i: (0, 0, 0)),
            vec(Cm), vec(Cm), vec(C2), vec(C2),
        ],
        out_specs=pl.BlockSpec((1, H * W, C2),
                               lambda p, i: (jnp.where(p == 2, i, 0), 0, 0)),
        out_shape=jax.ShapeDtypeStruct((N, H * W, C2), jnp.float32),
        scratch_shapes=[
            pltpu.VMEM((N, H, W, Cm), _MM),
            pltpu.VMEM((N, H * W, C2), _MM),
            pltpu.VMEM((2, Cm), jnp.float32),
            pltpu.VMEM((2, C2), jnp.float32),
            pltpu.VMEM((H + 2, W, 3 * C1), _MM),
            pltpu.VMEM((H + 2, W, 3 * Cm), _MM),
            pltpu.VMEM((H, W + 2, Cm), _MM),
        ],
        compiler_params=pltpu.CompilerParams(
            dimension_semantics=("arbitrary", "arbitrary"),
            vmem_limit_bytes=60 * 1024 * 1024),
    )(xp, w1c, w2c, g1r, b1r, g2r, b2r)

    return jnp.transpose(out.reshape(N, H, W, C2), (0, 3, 1, 2))


def kernel(x_nchw, w1, g1, b1, w2, g2, b2):
    return _double_conv(x_nchw, w1, g1, b1, w2, g2, b2)
